# baseline (speedup 1.0000x reference)
import functools

import jax
import jax.numpy as jnp
import numpy as np
from jax import lax
from jax.experimental import pallas as pl
from jax.experimental.pallas import tpu as pltpu
from jax.experimental.pallas import tpu_sc as plsc

N_NODE = 50000
E = 300000
HC = 128
HEADS = 8
DH = 16

N_PAD = 50176
E_PAD = 311296
EW = E_PAD // 16
NBLK = E_PAD // 128
NPASS = 8
RANGE = N_PAD // NPASS
ACC_ROWS = RANGE + 16
DEN_ROWS = N_PAD // 8


def _block_diag(a):
    out = jnp.zeros((HC, HC), jnp.float32)
    for h in range(HEADS):
        out = out.at[h * DH:(h + 1) * DH, h * DH:(h + 1) * DH].set(a[h])
    return out


def _proj_body(x_ref, wi_ref, bi_ref, wq_ref, bq_ref, wk_ref, bk_ref,
               wv_ref, bv_ref, xn_ref, q_ref, k_ref, v_ref):
    xn = jnp.dot(x_ref[...], wi_ref[...], preferred_element_type=jnp.float32)
    xn = xn + bi_ref[...]
    xn_ref[...] = xn
    q_ref[...] = jnp.dot(xn, wq_ref[...], preferred_element_type=jnp.float32) + bq_ref[...]
    k_ref[...] = jnp.dot(xn, wk_ref[...], preferred_element_type=jnp.float32) + bk_ref[...]
    v_ref[...] = jnp.dot(xn, wv_ref[...], preferred_element_type=jnp.float32) + bv_ref[...]


def _proj(x, wi, bi, wq, bq, wk, bk, wv, bv):
    row = pl.BlockSpec((512, HC), lambda i: (i, 0))
    wsp = pl.BlockSpec((HC, HC), lambda i: (0, 0))
    bsp = pl.BlockSpec((1, HC), lambda i: (0, 0))
    out = jax.ShapeDtypeStruct((N_PAD, HC), jnp.float32)
    return pl.pallas_call(
        _proj_body,
        grid=(N_PAD // 512,),
        in_specs=[row, wsp, bsp, wsp, bsp, wsp, bsp, wsp, bsp],
        out_specs=[row, row, row, row],
        out_shape=[out, out, out, out],
    )(x, wi, bi, wq, bq, wk, bk, wv, bv)


def _s2_body(k_ref, q_ref, v_ref, d8_ref, s16_ref, r8_ref, w_ref, exw_ref):
    prod = k_ref[...] * q_ref[...]
    ex16 = jnp.exp(jnp.dot(prod, s16_ref[...],
                           preferred_element_type=jnp.float32))
    ex_t = jnp.dot(ex16, r8_ref[...], preferred_element_type=jnp.float32)
    w_ref[...] = v_ref[...] * ex_t
    colg = lax.broadcasted_iota(jnp.int32, (512, HC), 1) // DH
    exw_ref[...] = jnp.where(colg == d8_ref[...], ex_t, 0.0)


def _s2(ke, qe, ve, d8, s16, r8):
    row = pl.BlockSpec((512, HC), lambda i: (i, 0))
    return pl.pallas_call(
        _s2_body,
        grid=(E_PAD // 512,),
        in_specs=[row, row, row,
                  pl.BlockSpec((512, 1), lambda i: (i, 0)),
                  pl.BlockSpec((HC, DH), lambda i: (0, 0)),
                  pl.BlockSpec((DH, HC), lambda i: (0, 0))],
        out_specs=[row, row],
        out_shape=[jax.ShapeDtypeStruct((E_PAD, HC), jnp.float32),
                   jax.ShapeDtypeStruct((E_PAD, HC), jnp.float32)],
    )(ke, qe, ve, d8, s16, r8)


def _post_body(agg0_ref, agg1_ref, den0_ref, den1_ref, xn_ref, wo_ref,
               bo_ref, r8_ref, blend_ref, prelu_ref, o_ref):
    den = den0_ref[...] + den1_ref[...]
    dw = jnp.dot(den, r8_ref[...], preferred_element_type=jnp.float32)
    a = (agg0_ref[...] + agg1_ref[...]) / (dw + 1e-16)
    g = jax.nn.gelu(a)
    o = jnp.dot(g, wo_ref[...], preferred_element_type=jnp.float32) + bo_ref[...]
    b = blend_ref[0, 0]
    o = b * o + (1.0 - b) * xn_ref[...]
    o_ref[...] = jnp.where(o > 0, o, prelu_ref[...] * o)


def _post(agg0, agg1, den0, den1, xn, wo, bo, r8, blend, prelu):
    row = pl.BlockSpec((512, HC), lambda i: (i, 0))
    wsp = pl.BlockSpec((HC, HC), lambda i: (0, 0))
    bsp = pl.BlockSpec((1, HC), lambda i: (0, 0))
    dsp = pl.BlockSpec((512, DH), lambda i: (i, 0))
    return pl.pallas_call(
        _post_body,
        grid=(N_PAD // 512,),
        in_specs=[row, row, dsp, dsp, row, wsp, bsp,
                  pl.BlockSpec((DH, HC), lambda i: (0, 0)),
                  pl.BlockSpec((1, 1), lambda i: (0, 0)),
                  bsp],
        out_specs=row,
        out_shape=jax.ShapeDtypeStruct((N_PAD, HC), jnp.float32),
    )(agg0, agg1, den0, den1, xn, wo, bo, r8, blend, prelu)


def _s1(ktbl_in, qtbl_in, vtbl_in, src_in, dst_in):
    mesh = plsc.VectorSubcoreMesh(core_axis_name="c", subcore_axis_name="s")
    eshape = jax.ShapeDtypeStruct((E_PAD, HC), jnp.float32)
    EW2 = E_PAD // 32

    NCH = EW2 // 256

    @functools.partial(
        pl.kernel,
        out_type=[eshape] * 3,
        mesh=mesh,
        scratch_types=[pltpu.VMEM((4, 128), jnp.int32),
                       pltpu.VMEM((4, 128), jnp.int32),
                       pltpu.VMEM((256, HC), jnp.float32),
                       pltpu.VMEM((256, HC), jnp.float32),
                       pltpu.VMEM((256, HC), jnp.float32),
                       pltpu.SemaphoreType.DMA,
                       pltpu.SemaphoreType.DMA,
                       pltpu.SemaphoreType.DMA],
    )
    def k(ktbl, qtbl, vtbl, src1d, dst1d, ke, qe, ve,
          sidx, didx, kbuf, qbuf, vbuf, isem, gsem, wsem):
        wid = lax.axis_index("s") * 2 + lax.axis_index("c")
        base = wid * EW2

        def chunk(ch, carry):
            eb = base + ch * 256
            icp = []
            for j in range(2):
                icp.append(pltpu.async_copy(
                    src1d.at[pl.ds(eb + j * 128, 128)], sidx.at[j], isem))
                icp.append(pltpu.async_copy(
                    dst1d.at[pl.ds(eb + j * 128, 128)], didx.at[j], isem))
            for cp in icp:
                cp.wait()
            cps = []
            for j in range(2):
                blk = pl.ds(j * 128, 128)
                cps.append(pltpu.async_copy(ktbl.at[sidx.at[j]],
                                            kbuf.at[blk], gsem))
                cps.append(pltpu.async_copy(qtbl.at[didx.at[j]],
                                            qbuf.at[blk], gsem))
                cps.append(pltpu.async_copy(vtbl.at[sidx.at[j]],
                                            vbuf.at[blk], gsem))
            for cp in cps:
                cp.wait()
            ocp = [pltpu.async_copy(kbuf, ke.at[pl.ds(eb, 256)], wsem),
                   pltpu.async_copy(qbuf, qe.at[pl.ds(eb, 256)], wsem),
                   pltpu.async_copy(vbuf, ve.at[pl.ds(eb, 256)], wsem)]
            for cp in ocp:
                cp.wait()
            return carry
        lax.fori_loop(0, NCH, chunk, 0)

    return k(ktbl_in, qtbl_in, vtbl_in, src_in, dst_in)


def _s3(w_h, exw_h, il_h, d3_h, bounds_h, zacc_h):
    mesh = plsc.VectorSubcoreMesh(core_axis_name="c", subcore_axis_name="s")

    @functools.partial(
        pl.kernel,
        out_type=[jax.ShapeDtypeStruct((2, N_PAD, HC), jnp.float32),
                  jax.ShapeDtypeStruct((2, DEN_ROWS, HC), jnp.float32)],
        mesh=mesh,
        scratch_types=[pltpu.VMEM((1, 128), jnp.int32),
                       pltpu.VMEM((4, 128), jnp.int32),
                       pltpu.VMEM((512, HC), jnp.float32),
                       pltpu.VMEM_SHARED((ACC_ROWS, HC), jnp.float32),
                       pltpu.SemaphoreType.DMA,
                       pltpu.SemaphoreType.DMA],
    )
    def k(w_in, exw_in, il_in, d3_in, bounds, zacc,
          agg_out, den_out,
          bbuf, ibuf, wbuf, acc_sp, gsem, asem):
        c = lax.axis_index("c")
        s = lax.axis_index("s")
        wid = s * 2 + c
        pltpu.sync_copy(bounds, bbuf)
        bv = bbuf[0, pl.ds(0, 16)]

        for p in range(NPASS):
            pltpu.sync_copy(zacc,
                            acc_sp.at[pl.ds(s * (ACC_ROWS // 16), ACC_ROWS // 16)])
            plsc.subcore_barrier()
            b0 = bv[p] // 128
            b1 = (bv[p + 1] + 127) // 128
            nloop = (b1 - b0 + 127) // 128

            def kiter(ki, carry):
                for j in range(4):
                    blk = b0 + wid * 4 + ki * 128 + j

                    @pl.when(blk < b1)
                    def _():
                        cps = [pltpu.async_copy(
                                   w_in.at[pl.ds(blk * 128, 128)],
                                   wbuf.at[pl.ds(j * 128, 128)], gsem),
                               pltpu.async_copy(
                                   il_in.at[p].at[pl.ds(blk * 128, 128)],
                                   ibuf.at[j], gsem)]
                        for cp in cps:
                            cp.wait()
                        pltpu.async_copy(wbuf.at[pl.ds(j * 128, 128)],
                                         acc_sp.at[ibuf.at[j]], asem,
                                         add=True).wait()
                return carry
            lax.fori_loop(0, nloop, kiter, 0)
            plsc.subcore_barrier()

            @pl.when(s == 0)
            def _():
                pltpu.sync_copy(acc_sp.at[pl.ds(0, RANGE)],
                                agg_out.at[c].at[pl.ds(p * RANGE, RANGE)])
            plsc.subcore_barrier()

        pltpu.sync_copy(zacc.at[pl.ds(0, DEN_ROWS // 16)],
                        acc_sp.at[pl.ds(s * (DEN_ROWS // 16), DEN_ROWS // 16)])
        plsc.subcore_barrier()

        def dchunk(ch, carry):
            for j in range(4):
                blk = wid * (NBLK // 32) + ch * 4 + j
                cps = [pltpu.async_copy(exw_in.at[pl.ds(blk * 128, 128)],
                                        wbuf.at[pl.ds(j * 128, 128)], gsem),
                       pltpu.async_copy(d3_in.at[pl.ds(blk * 128, 128)],
                                        ibuf.at[j], gsem)]
                for cp in cps:
                    cp.wait()
                pltpu.async_copy(wbuf.at[pl.ds(j * 128, 128)],
                                 acc_sp.at[ibuf.at[j]], asem, add=True).wait()
            return carry
        lax.fori_loop(0, NBLK // 32 // 4, dchunk, 0)
        plsc.subcore_barrier()

        @pl.when(s == 0)
        def _():
            pltpu.sync_copy(acc_sp.at[pl.ds(0, DEN_ROWS)], den_out.at[c])

    return k(w_h, exw_h, il_h, d3_h, bounds_h, zacc_h)


def _edge_prep(edge):
    src = jnp.concatenate([edge[0], jnp.full((E_PAD - E,), N_NODE, jnp.int32)])
    dst = jnp.concatenate([edge[1], jnp.full((E_PAD - E,), N_NODE, jnp.int32)])
    perm = jnp.argsort(dst // RANGE, stable=True)
    src = src[perm]
    dst = dst[perm]
    counts = jnp.bincount(dst // RANGE, length=NPASS)
    starts = jnp.concatenate([jnp.zeros((1,), counts.dtype),
                              jnp.cumsum(counts)]).astype(jnp.int32)
    bounds = jnp.zeros((1, 128), jnp.int32).at[0, :NPASS + 1].set(starts)
    il = jnp.stack([
        jnp.where((dst >= p * RANGE) & (dst < (p + 1) * RANGE),
                  dst - p * RANGE, RANGE).astype(jnp.int32)
        for p in range(NPASS)])
    d3 = (dst // 8).astype(jnp.int32)
    d8 = (dst % 8).astype(jnp.int32).reshape(E_PAD, 1)
    return src, dst, il, d3, d8, bounds


def kernel(x_paper, x_author, edge_index_p2a, edge_index_a2p,
           W_in_p, b_in_p, W_in_a, b_in_a,
           Wk_p, bk_p, Wq_p, bq_p, Wv_p, bv_p,
           Wk_a, bk_a, Wq_a, bq_a, Wv_a, bv_a,
           a_p2a, m_p2a, p_p2a, a_a2p, m_a2p, p_a2p,
           Wo_p, bo_p, Wo_a, bo_a, skip_p, skip_a, prelu_w):
    f32 = jnp.float32

    A_p2a = _block_diag(a_p2a); M_p2a = _block_diag(m_p2a)
    A_a2p = _block_diag(a_a2p); M_a2p = _block_diag(m_a2p)
    s_p2a = jnp.repeat(p_p2a, DH) / np.sqrt(DH)
    s_a2p = jnp.repeat(p_a2p, DH) / np.sqrt(DH)

    def pad_rows(x):
        return jnp.concatenate([x, jnp.zeros((N_PAD - N_NODE, HC), f32)], 0)

    b2 = lambda b: b.reshape(1, HC)
    xn_p, q_p, k_p, v_p = _proj(
        pad_rows(x_paper), W_in_p.T, b2(b_in_p),
        Wq_p.T * s_a2p[None, :], b2(bq_p * s_a2p),
        Wk_p.T @ A_p2a, b2(bk_p @ A_p2a),
        Wv_p.T @ M_p2a, b2(bv_p @ M_p2a))
    xn_a, q_a, k_a, v_a = _proj(
        pad_rows(x_author), W_in_a.T, b2(b_in_a),
        Wq_a.T * s_p2a[None, :], b2(bq_a * s_p2a),
        Wk_a.T @ A_a2p, b2(bk_a @ A_a2p),
        Wv_a.T @ M_a2p, b2(bv_a @ M_a2p))

    sp, dp, il_p, d3_p, d8_p, bd_p = _edge_prep(edge_index_p2a)
    sa, da, il_a, d3_a, d8_a, bd_a = _edge_prep(edge_index_a2p)

    s16 = jnp.zeros((HC, DH), f32)
    r8 = jnp.zeros((DH, HC), f32)
    for h in range(HEADS):
        s16 = s16.at[h * DH:(h + 1) * DH, h].set(1.0)
        r8 = r8.at[h, h * DH:(h + 1) * DH].set(1.0)
    zacc = jnp.zeros((ACC_ROWS // 16, HC), f32)

    ke_p, qe_p, ve_p = _s1(k_p, q_a, v_p, sp, dp)
    w_p2a, exw_p2a = _s2(ke_p, qe_p, ve_p, d8_p, s16, r8)
    ke_a, qe_a, ve_a = _s1(k_a, q_p, v_a, sa, da)
    w_a2p, exw_a2p = _s2(ke_a, qe_a, ve_a, d8_a, s16, r8)

    agg_a2, den_a2 = _s3(w_p2a, exw_p2a, il_p, d3_p, bd_p, zacc)
    agg_p2, den_p2 = _s3(w_a2p, exw_a2p, il_a, d3_a, bd_a, zacc)

    bp = jax.nn.sigmoid(skip_p).reshape(1, 1)
    ba = jax.nn.sigmoid(skip_a).reshape(1, 1)
    out_p = _post(agg_p2[0], agg_p2[1],
                  den_p2[0].reshape(N_PAD, DH), den_p2[1].reshape(N_PAD, DH),
                  xn_p, Wo_p.T, b2(bo_p), r8, bp, b2(prelu_w))
    out_a = _post(agg_a2[0], agg_a2[1],
                  den_a2[0].reshape(N_PAD, DH), den_a2[1].reshape(N_PAD, DH),
                  xn_a, Wo_a.T, b2(bo_a), r8, ba, b2(prelu_w))
    return out_p[:N_NODE], out_a[:N_NODE]


# speedup vs baseline: 1.1669x; 1.1669x over previous
import functools

import jax
import jax.numpy as jnp
import numpy as np
from jax import lax
from jax.experimental import pallas as pl
from jax.experimental.pallas import tpu as pltpu
from jax.experimental.pallas import tpu_sc as plsc

N_NODE = 50000
E = 300000
HC = 128
HEADS = 8
DH = 16

N_PAD = 50176
E_PAD = 307200
EW = E_PAD // 16
NBLK = E_PAD // 128
NPASS = 8
RANGE = N_PAD // NPASS
ACC_ROWS = RANGE + 16
DEN_ROWS = N_PAD // 8


def _block_diag(a):
    out = jnp.zeros((HC, HC), jnp.float32)
    for h in range(HEADS):
        out = out.at[h * DH:(h + 1) * DH, h * DH:(h + 1) * DH].set(a[h])
    return out


def _proj_body(x_ref, wi_ref, bi_ref, wq_ref, bq_ref, wk_ref, bk_ref,
               wv_ref, bv_ref, xn_ref, q_ref, k_ref, v_ref):
    xn = jnp.dot(x_ref[...], wi_ref[...], preferred_element_type=jnp.float32)
    xn = xn + bi_ref[...]
    xn_ref[...] = xn
    q_ref[...] = jnp.dot(xn, wq_ref[...], preferred_element_type=jnp.float32) + bq_ref[...]
    k_ref[...] = jnp.dot(xn, wk_ref[...], preferred_element_type=jnp.float32) + bk_ref[...]
    v_ref[...] = jnp.dot(xn, wv_ref[...], preferred_element_type=jnp.float32) + bv_ref[...]


def _proj(x, wi, bi, wq, bq, wk, bk, wv, bv):
    row = pl.BlockSpec((512, HC), lambda i: (i, 0))
    wsp = pl.BlockSpec((HC, HC), lambda i: (0, 0))
    bsp = pl.BlockSpec((1, HC), lambda i: (0, 0))
    out = jax.ShapeDtypeStruct((N_PAD, HC), jnp.float32)
    return pl.pallas_call(
        _proj_body,
        grid=(N_PAD // 512,),
        in_specs=[row, wsp, bsp, wsp, bsp, wsp, bsp, wsp, bsp],
        out_specs=[row, row, row, row],
        out_shape=[out, out, out, out],
    )(x, wi, bi, wq, bq, wk, bk, wv, bv)


def _s2_body(k_ref, q_ref, v_ref, d8_ref, s16_ref, r8_ref, w_ref, exw_ref):
    prod = k_ref[...] * q_ref[...]
    ex16 = jnp.exp(jnp.dot(prod, s16_ref[...],
                           preferred_element_type=jnp.float32))
    ex_t = jnp.dot(ex16, r8_ref[...], preferred_element_type=jnp.float32)
    w_ref[...] = v_ref[...] * ex_t
    colg = lax.broadcasted_iota(jnp.int32, (512, HC), 1) // DH
    exw_ref[...] = jnp.where(colg == d8_ref[...], ex_t, 0.0)


def _s2(ke, qe, ve, d8, s16, r8):
    row = pl.BlockSpec((512, HC), lambda i: (i, 0))
    return pl.pallas_call(
        _s2_body,
        grid=(E_PAD // 512,),
        in_specs=[row, row, row,
                  pl.BlockSpec((512, 1), lambda i: (i, 0)),
                  pl.BlockSpec((HC, DH), lambda i: (0, 0)),
                  pl.BlockSpec((DH, HC), lambda i: (0, 0))],
        out_specs=[row, row],
        out_shape=[jax.ShapeDtypeStruct((E_PAD, HC), jnp.float32),
                   jax.ShapeDtypeStruct((E_PAD, HC), jnp.float32)],
    )(ke, qe, ve, d8, s16, r8)


def _post_body(agg0_ref, agg1_ref, den0_ref, den1_ref, xn_ref, wo_ref,
               bo_ref, r8_ref, blend_ref, prelu_ref, o_ref):
    den = den0_ref[...] + den1_ref[...]
    dw = jnp.dot(den, r8_ref[...], preferred_element_type=jnp.float32)
    a = (agg0_ref[...] + agg1_ref[...]) / (dw + 1e-16)
    g = jax.nn.gelu(a)
    o = jnp.dot(g, wo_ref[...], preferred_element_type=jnp.float32) + bo_ref[...]
    b = blend_ref[0, 0]
    o = b * o + (1.0 - b) * xn_ref[...]
    o_ref[...] = jnp.where(o > 0, o, prelu_ref[...] * o)


def _post(agg0, agg1, den0, den1, xn, wo, bo, r8, blend, prelu):
    row = pl.BlockSpec((512, HC), lambda i: (i, 0))
    wsp = pl.BlockSpec((HC, HC), lambda i: (0, 0))
    bsp = pl.BlockSpec((1, HC), lambda i: (0, 0))
    dsp = pl.BlockSpec((512, DH), lambda i: (i, 0))
    return pl.pallas_call(
        _post_body,
        grid=(N_PAD // 512,),
        in_specs=[row, row, dsp, dsp, row, wsp, bsp,
                  pl.BlockSpec((DH, HC), lambda i: (0, 0)),
                  pl.BlockSpec((1, 1), lambda i: (0, 0)),
                  bsp],
        out_specs=row,
        out_shape=jax.ShapeDtypeStruct((N_PAD, HC), jnp.float32),
    )(agg0, agg1, den0, den1, xn, wo, bo, r8, blend, prelu)


def _s1(ktbl_in, qtbl_in, vtbl_in, src_in, dst_in):
    mesh = plsc.VectorSubcoreMesh(core_axis_name="c", subcore_axis_name="s")
    eshape = jax.ShapeDtypeStruct((E_PAD, HC), jnp.float32)
    EW2 = E_PAD // 32

    NCH = EW2 // 256

    @functools.partial(
        pl.kernel,
        out_type=[eshape] * 3,
        mesh=mesh,
        scratch_types=[pltpu.VMEM((4, 128), jnp.int32),
                       pltpu.VMEM((4, 128), jnp.int32),
                       pltpu.VMEM((256, HC), jnp.float32),
                       pltpu.VMEM((256, HC), jnp.float32),
                       pltpu.VMEM((256, HC), jnp.float32),
                       pltpu.SemaphoreType.DMA,
                       pltpu.SemaphoreType.DMA,
                       pltpu.SemaphoreType.DMA],
    )
    def k(ktbl, qtbl, vtbl, src1d, dst1d, ke, qe, ve,
          sidx, didx, kbuf, qbuf, vbuf, isem, gsem, wsem):
        wid = lax.axis_index("s") * 2 + lax.axis_index("c")
        base = wid * EW2

        def chunk(ch, carry):
            eb = base + ch * 256
            icp = []
            for j in range(2):
                icp.append(pltpu.async_copy(
                    src1d.at[pl.ds(eb + j * 128, 128)], sidx.at[j], isem))
                icp.append(pltpu.async_copy(
                    dst1d.at[pl.ds(eb + j * 128, 128)], didx.at[j], isem))
            for cp in icp:
                cp.wait()
            cps = []
            for j in range(2):
                blk = pl.ds(j * 128, 128)
                cps.append(pltpu.async_copy(ktbl.at[sidx.at[j]],
                                            kbuf.at[blk], gsem))
                cps.append(pltpu.async_copy(qtbl.at[didx.at[j]],
                                            qbuf.at[blk], gsem))
                cps.append(pltpu.async_copy(vtbl.at[sidx.at[j]],
                                            vbuf.at[blk], gsem))
            for cp in cps:
                cp.wait()
            ocp = [pltpu.async_copy(kbuf, ke.at[pl.ds(eb, 256)], wsem),
                   pltpu.async_copy(qbuf, qe.at[pl.ds(eb, 256)], wsem),
                   pltpu.async_copy(vbuf, ve.at[pl.ds(eb, 256)], wsem)]
            for cp in ocp:
                cp.wait()
            return carry
        lax.fori_loop(0, EW2 // 256, chunk, 0)

        eb = base + (EW2 // 256) * 256
        icp = [pltpu.async_copy(src1d.at[pl.ds(eb, 128)], sidx.at[0], isem),
               pltpu.async_copy(dst1d.at[pl.ds(eb, 128)], didx.at[0], isem)]
        for cp in icp:
            cp.wait()
        blk = pl.ds(0, 128)
        cps = [pltpu.async_copy(ktbl.at[sidx.at[0]], kbuf.at[blk], gsem),
               pltpu.async_copy(qtbl.at[didx.at[0]], qbuf.at[blk], gsem),
               pltpu.async_copy(vtbl.at[sidx.at[0]], vbuf.at[blk], gsem)]
        for cp in cps:
            cp.wait()
        ocp = [pltpu.async_copy(kbuf.at[blk], ke.at[pl.ds(eb, 128)], wsem),
               pltpu.async_copy(qbuf.at[blk], qe.at[pl.ds(eb, 128)], wsem),
               pltpu.async_copy(vbuf.at[blk], ve.at[pl.ds(eb, 128)], wsem)]
        for cp in ocp:
            cp.wait()

    return k(ktbl_in, qtbl_in, vtbl_in, src_in, dst_in)


def _s3(w_h, exw_h, il_h, d3_h, bounds_h, zacc_h):
    mesh = plsc.VectorSubcoreMesh(core_axis_name="c", subcore_axis_name="s")

    @functools.partial(
        pl.kernel,
        out_type=[jax.ShapeDtypeStruct((2, N_PAD, HC), jnp.float32),
                  jax.ShapeDtypeStruct((2, DEN_ROWS, HC), jnp.float32)],
        mesh=mesh,
        scratch_types=[pltpu.VMEM((1, 128), jnp.int32),
                       pltpu.VMEM((4, 128), jnp.int32),
                       pltpu.VMEM((512, HC), jnp.float32),
                       pltpu.VMEM_SHARED((ACC_ROWS, HC), jnp.float32),
                       pltpu.SemaphoreType.DMA,
                       pltpu.SemaphoreType.DMA],
    )
    def k(w_in, exw_in, il_in, d3_in, bounds, zacc,
          agg_out, den_out,
          bbuf, ibuf, wbuf, acc_sp, gsem, asem):
        c = lax.axis_index("c")
        s = lax.axis_index("s")
        wid = s * 2 + c
        pltpu.sync_copy(bounds, bbuf)
        bv = bbuf[0, pl.ds(0, 16)]

        for p in range(NPASS):
            pltpu.sync_copy(zacc,
                            acc_sp.at[pl.ds(s * (ACC_ROWS // 16), ACC_ROWS // 16)])
            plsc.subcore_barrier()
            b0 = bv[p] // 128
            b1 = (bv[p + 1] + 127) // 128
            nloop = (b1 - b0 + 127) // 128

            def kiter(ki, carry):
                for j in range(4):
                    blk = b0 + wid * 4 + ki * 128 + j

                    @pl.when(blk < b1)
                    def _():
                        cps = [pltpu.async_copy(
                                   w_in.at[pl.ds(blk * 128, 128)],
                                   wbuf.at[pl.ds(j * 128, 128)], gsem),
                               pltpu.async_copy(
                                   il_in.at[p].at[pl.ds(blk * 128, 128)],
                                   ibuf.at[j], gsem)]
                        for cp in cps:
                            cp.wait()
                        pltpu.async_copy(wbuf.at[pl.ds(j * 128, 128)],
                                         acc_sp.at[ibuf.at[j]], asem,
                                         add=True).wait()
                return carry
            lax.fori_loop(0, nloop, kiter, 0)
            plsc.subcore_barrier()

            @pl.when(s == 0)
            def _():
                pltpu.sync_copy(acc_sp.at[pl.ds(0, RANGE)],
                                agg_out.at[c].at[pl.ds(p * RANGE, RANGE)])
            plsc.subcore_barrier()

        pltpu.sync_copy(zacc.at[pl.ds(0, DEN_ROWS // 16)],
                        acc_sp.at[pl.ds(s * (DEN_ROWS // 16), DEN_ROWS // 16)])
        plsc.subcore_barrier()

        def dchunk(ch, carry):
            for j in range(3):
                blk = wid * (NBLK // 32) + ch * 3 + j
                cps = [pltpu.async_copy(exw_in.at[pl.ds(blk * 128, 128)],
                                        wbuf.at[pl.ds(j * 128, 128)], gsem),
                       pltpu.async_copy(d3_in.at[pl.ds(blk * 128, 128)],
                                        ibuf.at[j], gsem)]
                for cp in cps:
                    cp.wait()
                pltpu.async_copy(wbuf.at[pl.ds(j * 128, 128)],
                                 acc_sp.at[ibuf.at[j]], asem, add=True).wait()
            return carry
        lax.fori_loop(0, NBLK // 32 // 3, dchunk, 0)
        plsc.subcore_barrier()

        @pl.when(s == 0)
        def _():
            pltpu.sync_copy(acc_sp.at[pl.ds(0, DEN_ROWS)], den_out.at[c])

    return k(w_h, exw_h, il_h, d3_h, bounds_h, zacc_h)


def _edge_prep(edge):
    src = jnp.concatenate([edge[0], jnp.full((E_PAD - E,), N_NODE, jnp.int32)])
    dst = jnp.concatenate([edge[1], jnp.full((E_PAD - E,), N_NODE, jnp.int32)])
    perm = jnp.argsort(dst // RANGE, stable=True)
    src = src[perm]
    dst = dst[perm]
    counts = jnp.bincount(dst // RANGE, length=NPASS)
    starts = jnp.concatenate([jnp.zeros((1,), counts.dtype),
                              jnp.cumsum(counts)]).astype(jnp.int32)
    bounds = jnp.zeros((1, 128), jnp.int32).at[0, :NPASS + 1].set(starts)
    il = jnp.stack([
        jnp.where((dst >= p * RANGE) & (dst < (p + 1) * RANGE),
                  dst - p * RANGE, RANGE).astype(jnp.int32)
        for p in range(NPASS)])
    d3 = (dst // 8).astype(jnp.int32)
    d8 = (dst % 8).astype(jnp.int32).reshape(E_PAD, 1)
    return src, dst, il, d3, d8, bounds


def kernel(x_paper, x_author, edge_index_p2a, edge_index_a2p,
           W_in_p, b_in_p, W_in_a, b_in_a,
           Wk_p, bk_p, Wq_p, bq_p, Wv_p, bv_p,
           Wk_a, bk_a, Wq_a, bq_a, Wv_a, bv_a,
           a_p2a, m_p2a, p_p2a, a_a2p, m_a2p, p_a2p,
           Wo_p, bo_p, Wo_a, bo_a, skip_p, skip_a, prelu_w):
    f32 = jnp.float32

    A_p2a = _block_diag(a_p2a); M_p2a = _block_diag(m_p2a)
    A_a2p = _block_diag(a_a2p); M_a2p = _block_diag(m_a2p)
    s_p2a = jnp.repeat(p_p2a, DH) / np.sqrt(DH)
    s_a2p = jnp.repeat(p_a2p, DH) / np.sqrt(DH)

    def pad_rows(x):
        return jnp.concatenate([x, jnp.zeros((N_PAD - N_NODE, HC), f32)], 0)

    b2 = lambda b: b.reshape(1, HC)
    xn_p, q_p, k_p, v_p = _proj(
        pad_rows(x_paper), W_in_p.T, b2(b_in_p),
        Wq_p.T * s_a2p[None, :], b2(bq_p * s_a2p),
        Wk_p.T @ A_p2a, b2(bk_p @ A_p2a),
        Wv_p.T @ M_p2a, b2(bv_p @ M_p2a))
    xn_a, q_a, k_a, v_a = _proj(
        pad_rows(x_author), W_in_a.T, b2(b_in_a),
        Wq_a.T * s_p2a[None, :], b2(bq_a * s_p2a),
        Wk_a.T @ A_a2p, b2(bk_a @ A_a2p),
        Wv_a.T @ M_a2p, b2(bv_a @ M_a2p))

    sp, dp, il_p, d3_p, d8_p, bd_p = _edge_prep(edge_index_p2a)
    sa, da, il_a, d3_a, d8_a, bd_a = _edge_prep(edge_index_a2p)

    s16 = jnp.zeros((HC, DH), f32)
    r8 = jnp.zeros((DH, HC), f32)
    for h in range(HEADS):
        s16 = s16.at[h * DH:(h + 1) * DH, h].set(1.0)
        r8 = r8.at[h, h * DH:(h + 1) * DH].set(1.0)
    zacc = jnp.zeros((ACC_ROWS // 16, HC), f32)

    ke_p, qe_p, ve_p = _s1(k_p, q_a, v_p, sp, dp)
    w_p2a, exw_p2a = _s2(ke_p, qe_p, ve_p, d8_p, s16, r8)
    ke_a, qe_a, ve_a = _s1(k_a, q_p, v_a, sa, da)
    w_a2p, exw_a2p = _s2(ke_a, qe_a, ve_a, d8_a, s16, r8)

    agg_a2, den_a2 = _s3(w_p2a, exw_p2a, il_p, d3_p, bd_p, zacc)
    agg_p2, den_p2 = _s3(w_a2p, exw_a2p, il_a, d3_a, bd_a, zacc)

    bp = jax.nn.sigmoid(skip_p).reshape(1, 1)
    ba = jax.nn.sigmoid(skip_a).reshape(1, 1)
    out_p = _post(agg_p2[0], agg_p2[1],
                  den_p2[0].reshape(N_PAD, DH), den_p2[1].reshape(N_PAD, DH),
                  xn_p, Wo_p.T, b2(bo_p), r8, bp, b2(prelu_w))
    out_a = _post(agg_a2[0], agg_a2[1],
                  den_a2[0].reshape(N_PAD, DH), den_a2[1].reshape(N_PAD, DH),
                  xn_a, Wo_a.T, b2(bo_a), r8, ba, b2(prelu_w))
    return out_p[:N_NODE], out_a[:N_NODE]


# speedup vs baseline: 1.3737x; 1.1773x over previous
import functools

import jax
import jax.numpy as jnp
import numpy as np
from jax import lax
from jax.experimental import pallas as pl
from jax.experimental.pallas import tpu as pltpu
from jax.experimental.pallas import tpu_sc as plsc

N_NODE = 50000
E = 300000
HC = 128
HEADS = 8
DH = 16

N_PAD = 50176
E_PAD = 303104
EW = E_PAD // 16
NBLK = E_PAD // 128
NPASS = 8
RANGE = N_PAD // NPASS
ACC_ROWS = RANGE + 16
DEN_ROWS = N_PAD // 8


def _block_diag(a):
    out = jnp.zeros((HC, HC), jnp.float32)
    for h in range(HEADS):
        out = out.at[h * DH:(h + 1) * DH, h * DH:(h + 1) * DH].set(a[h])
    return out


def _proj_body(x_ref, wi_ref, bi_ref, wq_ref, bq_ref, wk_ref, bk_ref,
               wv_ref, bv_ref, xn_ref, q_ref, k_ref, v_ref):
    xn = jnp.dot(x_ref[...], wi_ref[...], preferred_element_type=jnp.float32)
    xn = xn + bi_ref[...]
    xn_ref[...] = xn
    q_ref[...] = jnp.dot(xn, wq_ref[...], preferred_element_type=jnp.float32) + bq_ref[...]
    k_ref[...] = jnp.dot(xn, wk_ref[...], preferred_element_type=jnp.float32) + bk_ref[...]
    v_ref[...] = jnp.dot(xn, wv_ref[...], preferred_element_type=jnp.float32) + bv_ref[...]


def _proj(x, wi, bi, wq, bq, wk, bk, wv, bv):
    row = pl.BlockSpec((512, HC), lambda i: (i, 0))
    wsp = pl.BlockSpec((HC, HC), lambda i: (0, 0))
    bsp = pl.BlockSpec((1, HC), lambda i: (0, 0))
    out = jax.ShapeDtypeStruct((N_PAD, HC), jnp.float32)
    return pl.pallas_call(
        _proj_body,
        grid=(N_PAD // 512,),
        in_specs=[row, wsp, bsp, wsp, bsp, wsp, bsp, wsp, bsp],
        out_specs=[row, row, row, row],
        out_shape=[out, out, out, out],
    )(x, wi, bi, wq, bq, wk, bk, wv, bv)


def _s2_body(k_ref, q_ref, v_ref, d8_ref, s16_ref, r8_ref, w_ref, exw_ref):
    prod = k_ref[...] * q_ref[...]
    ex16 = jnp.exp(jnp.dot(prod, s16_ref[...],
                           preferred_element_type=jnp.float32))
    ex_t = jnp.dot(ex16, r8_ref[...], preferred_element_type=jnp.float32)
    w_ref[...] = v_ref[...] * ex_t
    colg = lax.broadcasted_iota(jnp.int32, (512, HC), 1) // DH
    exw_ref[...] = jnp.where(colg == d8_ref[...], ex_t, 0.0)


def _s2(ke, qe, ve, d8, s16, r8):
    row = pl.BlockSpec((512, HC), lambda i: (i, 0))
    return pl.pallas_call(
        _s2_body,
        grid=(E_PAD // 512,),
        in_specs=[row, row, row,
                  pl.BlockSpec((512, 1), lambda i: (i, 0)),
                  pl.BlockSpec((HC, DH), lambda i: (0, 0)),
                  pl.BlockSpec((DH, HC), lambda i: (0, 0))],
        out_specs=[row, row],
        out_shape=[jax.ShapeDtypeStruct((E_PAD, HC), jnp.float32),
                   jax.ShapeDtypeStruct((E_PAD, HC), jnp.float32)],
    )(ke, qe, ve, d8, s16, r8)


def _post_body(agg0_ref, agg1_ref, den0_ref, den1_ref, xn_ref, wo_ref,
               bo_ref, r8_ref, blend_ref, prelu_ref, o_ref):
    den = den0_ref[...] + den1_ref[...]
    dw = jnp.dot(den, r8_ref[...], preferred_element_type=jnp.float32)
    a = (agg0_ref[...] + agg1_ref[...]) / (dw + 1e-16)
    g = jax.nn.gelu(a)
    o = jnp.dot(g, wo_ref[...], preferred_element_type=jnp.float32) + bo_ref[...]
    b = blend_ref[0, 0]
    o = b * o + (1.0 - b) * xn_ref[...]
    o_ref[...] = jnp.where(o > 0, o, prelu_ref[...] * o)


def _post(agg0, agg1, den0, den1, xn, wo, bo, r8, blend, prelu):
    row = pl.BlockSpec((512, HC), lambda i: (i, 0))
    wsp = pl.BlockSpec((HC, HC), lambda i: (0, 0))
    bsp = pl.BlockSpec((1, HC), lambda i: (0, 0))
    dsp = pl.BlockSpec((512, DH), lambda i: (i, 0))
    return pl.pallas_call(
        _post_body,
        grid=(N_PAD // 512,),
        in_specs=[row, row, dsp, dsp, row, wsp, bsp,
                  pl.BlockSpec((DH, HC), lambda i: (0, 0)),
                  pl.BlockSpec((1, 1), lambda i: (0, 0)),
                  bsp],
        out_specs=row,
        out_shape=jax.ShapeDtypeStruct((N_PAD, HC), jnp.float32),
    )(agg0, agg1, den0, den1, xn, wo, bo, r8, blend, prelu)


def _s1(ktbl_in, qtbl_in, vtbl_in, src_in, dst_in):
    mesh = plsc.VectorSubcoreMesh(core_axis_name="c", subcore_axis_name="s")
    eshape = jax.ShapeDtypeStruct((E_PAD, HC), jnp.float32)
    EW2 = E_PAD // 32

    NCH = EW2 // 256

    @functools.partial(
        pl.kernel,
        out_type=[eshape] * 3,
        mesh=mesh,
        scratch_types=[pltpu.VMEM((4, 128), jnp.int32),
                       pltpu.VMEM((4, 128), jnp.int32),
                       pltpu.VMEM((256, HC), jnp.float32),
                       pltpu.VMEM((256, HC), jnp.float32),
                       pltpu.VMEM((256, HC), jnp.float32),
                       pltpu.SemaphoreType.DMA,
                       pltpu.SemaphoreType.DMA,
                       pltpu.SemaphoreType.DMA],
    )
    def k(ktbl, qtbl, vtbl, src1d, dst1d, ke, qe, ve,
          sidx, didx, kbuf, qbuf, vbuf, isem, gsem, wsem):
        wid = lax.axis_index("s") * 2 + lax.axis_index("c")
        base = wid * EW2

        def chunk(ch, carry):
            eb = base + ch * 256
            icp = []
            for j in range(2):
                icp.append(pltpu.async_copy(
                    src1d.at[pl.ds(eb + j * 128, 128)], sidx.at[j], isem))
                icp.append(pltpu.async_copy(
                    dst1d.at[pl.ds(eb + j * 128, 128)], didx.at[j], isem))
            for cp in icp:
                cp.wait()
            cps = []
            for j in range(2):
                blk = pl.ds(j * 128, 128)
                cps.append(pltpu.async_copy(ktbl.at[sidx.at[j]],
                                            kbuf.at[blk], gsem))
                cps.append(pltpu.async_copy(qtbl.at[didx.at[j]],
                                            qbuf.at[blk], gsem))
                cps.append(pltpu.async_copy(vtbl.at[sidx.at[j]],
                                            vbuf.at[blk], gsem))
            for cp in cps:
                cp.wait()
            ocp = [pltpu.async_copy(kbuf, ke.at[pl.ds(eb, 256)], wsem),
                   pltpu.async_copy(qbuf, qe.at[pl.ds(eb, 256)], wsem),
                   pltpu.async_copy(vbuf, ve.at[pl.ds(eb, 256)], wsem)]
            for cp in ocp:
                cp.wait()
            return carry
        lax.fori_loop(0, EW2 // 256, chunk, 0)

    return k(ktbl_in, qtbl_in, vtbl_in, src_in, dst_in)


def _s3(w_h, exw_h, il_h, d3_h, bounds_h, zacc_h):
    mesh = plsc.VectorSubcoreMesh(core_axis_name="c", subcore_axis_name="s")

    @functools.partial(
        pl.kernel,
        out_type=[jax.ShapeDtypeStruct((2, N_PAD, HC), jnp.float32),
                  jax.ShapeDtypeStruct((2, DEN_ROWS, HC), jnp.float32)],
        mesh=mesh,
        scratch_types=[pltpu.VMEM((1, 128), jnp.int32),
                       pltpu.VMEM((4, 128), jnp.int32),
                       pltpu.VMEM((512, HC), jnp.float32),
                       pltpu.VMEM_SHARED((ACC_ROWS, HC), jnp.float32),
                       pltpu.SemaphoreType.DMA,
                       pltpu.SemaphoreType.DMA],
    )
    def k(w_in, exw_in, il_in, d3_in, bounds, zacc,
          agg_out, den_out,
          bbuf, ibuf, wbuf, acc_sp, gsem, asem):
        c = lax.axis_index("c")
        s = lax.axis_index("s")
        wid = s * 2 + c
        pltpu.sync_copy(bounds, bbuf)
        bv = bbuf[0, pl.ds(0, 16)]

        for p in range(NPASS):
            pltpu.sync_copy(zacc,
                            acc_sp.at[pl.ds(s * (ACC_ROWS // 16), ACC_ROWS // 16)])
            plsc.subcore_barrier()
            b0 = bv[p] // 128
            b1 = (bv[p + 1] + 127) // 128
            nloop = (b1 - b0 + 127) // 128

            def kiter(ki, carry):
                for j in range(4):
                    blk = b0 + wid * 4 + ki * 128 + j

                    @pl.when(blk < b1)
                    def _():
                        cps = [pltpu.async_copy(
                                   w_in.at[pl.ds(blk * 128, 128)],
                                   wbuf.at[pl.ds(j * 128, 128)], gsem),
                               pltpu.async_copy(
                                   il_in.at[p].at[pl.ds(blk * 128, 128)],
                                   ibuf.at[j], gsem)]
                        for cp in cps:
                            cp.wait()
                        pltpu.async_copy(wbuf.at[pl.ds(j * 128, 128)],
                                         acc_sp.at[ibuf.at[j]], asem,
                                         add=True).wait()
                return carry
            lax.fori_loop(0, nloop, kiter, 0)
            plsc.subcore_barrier()

            @pl.when(s == 0)
            def _():
                pltpu.sync_copy(acc_sp.at[pl.ds(0, RANGE)],
                                agg_out.at[c].at[pl.ds(p * RANGE, RANGE)])
            plsc.subcore_barrier()

        pltpu.sync_copy(zacc.at[pl.ds(0, DEN_ROWS // 16)],
                        acc_sp.at[pl.ds(s * (DEN_ROWS // 16), DEN_ROWS // 16)])
        plsc.subcore_barrier()

        def dchunk(ch, carry):
            for j in range(2):
                blk = wid * (NBLK // 32) + ch * 2 + j
                cps = [pltpu.async_copy(exw_in.at[pl.ds(blk * 128, 128)],
                                        wbuf.at[pl.ds(j * 128, 128)], gsem),
                       pltpu.async_copy(d3_in.at[pl.ds(blk * 128, 128)],
                                        ibuf.at[j], gsem)]
                for cp in cps:
                    cp.wait()
                pltpu.async_copy(wbuf.at[pl.ds(j * 128, 128)],
                                 acc_sp.at[ibuf.at[j]], asem, add=True).wait()
            return carry
        lax.fori_loop(0, NBLK // 32 // 2, dchunk, 0)
        plsc.subcore_barrier()

        @pl.when(s == 0)
        def _():
            pltpu.sync_copy(acc_sp.at[pl.ds(0, DEN_ROWS)], den_out.at[c])

    return k(w_h, exw_h, il_h, d3_h, bounds_h, zacc_h)


def _edge_prep(edge):
    padi = N_NODE + jnp.arange(E_PAD - E, dtype=jnp.int32) % (N_PAD - N_NODE)
    src = jnp.concatenate([edge[0], padi])
    dst = jnp.concatenate([edge[1], padi])
    perm = jnp.argsort(dst // RANGE, stable=True)
    src = src[perm]
    dst = dst[perm]
    counts = jnp.bincount(dst // RANGE, length=NPASS)
    starts = jnp.concatenate([jnp.zeros((1,), counts.dtype),
                              jnp.cumsum(counts)]).astype(jnp.int32)
    bounds = jnp.zeros((1, 128), jnp.int32).at[0, :NPASS + 1].set(starts)
    il = jnp.stack([
        jnp.where((dst >= p * RANGE) & (dst < (p + 1) * RANGE),
                  dst - p * RANGE, RANGE).astype(jnp.int32)
        for p in range(NPASS)])
    d3 = (dst // 8).astype(jnp.int32)
    d8 = (dst % 8).astype(jnp.int32).reshape(E_PAD, 1)
    return src, dst, il, d3, d8, bounds


def kernel(x_paper, x_author, edge_index_p2a, edge_index_a2p,
           W_in_p, b_in_p, W_in_a, b_in_a,
           Wk_p, bk_p, Wq_p, bq_p, Wv_p, bv_p,
           Wk_a, bk_a, Wq_a, bq_a, Wv_a, bv_a,
           a_p2a, m_p2a, p_p2a, a_a2p, m_a2p, p_a2p,
           Wo_p, bo_p, Wo_a, bo_a, skip_p, skip_a, prelu_w):
    f32 = jnp.float32

    A_p2a = _block_diag(a_p2a); M_p2a = _block_diag(m_p2a)
    A_a2p = _block_diag(a_a2p); M_a2p = _block_diag(m_a2p)
    s_p2a = jnp.repeat(p_p2a, DH) / np.sqrt(DH)
    s_a2p = jnp.repeat(p_a2p, DH) / np.sqrt(DH)

    def pad_rows(x):
        return jnp.concatenate([x, jnp.zeros((N_PAD - N_NODE, HC), f32)], 0)

    b2 = lambda b: b.reshape(1, HC)
    xn_p, q_p, k_p, v_p = _proj(
        pad_rows(x_paper), W_in_p.T, b2(b_in_p),
        Wq_p.T * s_a2p[None, :], b2(bq_p * s_a2p),
        Wk_p.T @ A_p2a, b2(bk_p @ A_p2a),
        Wv_p.T @ M_p2a, b2(bv_p @ M_p2a))
    xn_a, q_a, k_a, v_a = _proj(
        pad_rows(x_author), W_in_a.T, b2(b_in_a),
        Wq_a.T * s_p2a[None, :], b2(bq_a * s_p2a),
        Wk_a.T @ A_a2p, b2(bk_a @ A_a2p),
        Wv_a.T @ M_a2p, b2(bv_a @ M_a2p))

    sp, dp, il_p, d3_p, d8_p, bd_p = _edge_prep(edge_index_p2a)
    sa, da, il_a, d3_a, d8_a, bd_a = _edge_prep(edge_index_a2p)

    s16 = jnp.zeros((HC, DH), f32)
    r8 = jnp.zeros((DH, HC), f32)
    for h in range(HEADS):
        s16 = s16.at[h * DH:(h + 1) * DH, h].set(1.0)
        r8 = r8.at[h, h * DH:(h + 1) * DH].set(1.0)
    zacc = jnp.zeros((ACC_ROWS // 16, HC), f32)

    ke_p, qe_p, ve_p = _s1(k_p, q_a, v_p, sp, dp)
    w_p2a, exw_p2a = _s2(ke_p, qe_p, ve_p, d8_p, s16, r8)
    ke_a, qe_a, ve_a = _s1(k_a, q_p, v_a, sa, da)
    w_a2p, exw_a2p = _s2(ke_a, qe_a, ve_a, d8_a, s16, r8)

    agg_a2, den_a2 = _s3(w_p2a, exw_p2a, il_p, d3_p, bd_p, zacc)
    agg_p2, den_p2 = _s3(w_a2p, exw_a2p, il_a, d3_a, bd_a, zacc)

    bp = jax.nn.sigmoid(skip_p).reshape(1, 1)
    ba = jax.nn.sigmoid(skip_a).reshape(1, 1)
    out_p = _post(agg_p2[0], agg_p2[1],
                  den_p2[0].reshape(N_PAD, DH), den_p2[1].reshape(N_PAD, DH),
                  xn_p, Wo_p.T, b2(bo_p), r8, bp, b2(prelu_w))
    out_a = _post(agg_a2[0], agg_a2[1],
                  den_a2[0].reshape(N_PAD, DH), den_a2[1].reshape(N_PAD, DH),
                  xn_a, Wo_a.T, b2(bo_a), r8, ba, b2(prelu_w))
    return out_p[:N_NODE], out_a[:N_NODE]


# speedup vs baseline: 1.4229x; 1.0358x over previous
import functools

import jax
import jax.numpy as jnp
import numpy as np
from jax import lax
from jax.experimental import pallas as pl
from jax.experimental.pallas import tpu as pltpu
from jax.experimental.pallas import tpu_sc as plsc

N_NODE = 50000
E = 300000
HC = 128
HEADS = 8
DH = 16

N_PAD = 50176
E_PAD = 303104
EW = E_PAD // 16
NBLK = E_PAD // 128
NPASS = 8
RANGE = N_PAD // NPASS
ACC_ROWS = RANGE + 16
DEN_ROWS = N_PAD // 8


def _block_diag(a):
    out = jnp.zeros((HC, HC), jnp.float32)
    for h in range(HEADS):
        out = out.at[h * DH:(h + 1) * DH, h * DH:(h + 1) * DH].set(a[h])
    return out


def _proj_body(x_ref, wi_ref, bi_ref, wq_ref, bq_ref, wk_ref, bk_ref,
               wv_ref, bv_ref, xn_ref, q_ref, k_ref, v_ref):
    xn = jnp.dot(x_ref[...], wi_ref[...], preferred_element_type=jnp.float32)
    xn = xn + bi_ref[...]
    xn_ref[...] = xn
    q_ref[...] = jnp.dot(xn, wq_ref[...], preferred_element_type=jnp.float32) + bq_ref[...]
    k_ref[...] = jnp.dot(xn, wk_ref[...], preferred_element_type=jnp.float32) + bk_ref[...]
    v_ref[...] = jnp.dot(xn, wv_ref[...], preferred_element_type=jnp.float32) + bv_ref[...]


def _proj(x, wi, bi, wq, bq, wk, bk, wv, bv):
    row = pl.BlockSpec((512, HC), lambda i: (i, 0))
    wsp = pl.BlockSpec((HC, HC), lambda i: (0, 0))
    bsp = pl.BlockSpec((1, HC), lambda i: (0, 0))
    out = jax.ShapeDtypeStruct((N_PAD, HC), jnp.float32)
    return pl.pallas_call(
        _proj_body,
        grid=(N_PAD // 512,),
        in_specs=[row, wsp, bsp, wsp, bsp, wsp, bsp, wsp, bsp],
        out_specs=[row, row, row, row],
        out_shape=[out, out, out, out],
    )(x, wi, bi, wq, bq, wk, bk, wv, bv)


def _s2_body(k_ref, q_ref, v_ref, d8_ref, s16_ref, r8_ref, w_ref, exw_ref):
    prod = k_ref[...] * q_ref[...]
    ex16 = jnp.exp(jnp.dot(prod, s16_ref[...],
                           preferred_element_type=jnp.float32))
    ex_t = jnp.dot(ex16, r8_ref[...], preferred_element_type=jnp.float32)
    w_ref[...] = v_ref[...] * ex_t
    colg = lax.broadcasted_iota(jnp.int32, (512, HC), 1) // DH
    exw_ref[...] = jnp.where(colg == d8_ref[...], ex_t, 0.0)


def _s2(ke, qe, ve, d8, s16, r8):
    row = pl.BlockSpec((512, HC), lambda i: (i, 0))
    return pl.pallas_call(
        _s2_body,
        grid=(E_PAD // 512,),
        in_specs=[row, row, row,
                  pl.BlockSpec((512, 1), lambda i: (i, 0)),
                  pl.BlockSpec((HC, DH), lambda i: (0, 0)),
                  pl.BlockSpec((DH, HC), lambda i: (0, 0))],
        out_specs=[row, row],
        out_shape=[jax.ShapeDtypeStruct((E_PAD, HC), jnp.float32),
                   jax.ShapeDtypeStruct((E_PAD, HC), jnp.float32)],
    )(ke, qe, ve, d8, s16, r8)


def _post_body(agg0_ref, agg1_ref, den0_ref, den1_ref, xn_ref, wo_ref,
               bo_ref, r8_ref, blend_ref, prelu_ref, o_ref):
    den = den0_ref[...] + den1_ref[...]
    dw = jnp.dot(den, r8_ref[...], preferred_element_type=jnp.float32)
    a = (agg0_ref[...] + agg1_ref[...]) / (dw + 1e-16)
    g = jax.nn.gelu(a)
    o = jnp.dot(g, wo_ref[...], preferred_element_type=jnp.float32) + bo_ref[...]
    b = blend_ref[0, 0]
    o = b * o + (1.0 - b) * xn_ref[...]
    o_ref[...] = jnp.where(o > 0, o, prelu_ref[...] * o)


def _post(agg0, agg1, den0, den1, xn, wo, bo, r8, blend, prelu):
    row = pl.BlockSpec((512, HC), lambda i: (i, 0))
    wsp = pl.BlockSpec((HC, HC), lambda i: (0, 0))
    bsp = pl.BlockSpec((1, HC), lambda i: (0, 0))
    dsp = pl.BlockSpec((512, DH), lambda i: (i, 0))
    return pl.pallas_call(
        _post_body,
        grid=(N_PAD // 512,),
        in_specs=[row, row, dsp, dsp, row, wsp, bsp,
                  pl.BlockSpec((DH, HC), lambda i: (0, 0)),
                  pl.BlockSpec((1, 1), lambda i: (0, 0)),
                  bsp],
        out_specs=row,
        out_shape=jax.ShapeDtypeStruct((N_PAD, HC), jnp.float32),
    )(agg0, agg1, den0, den1, xn, wo, bo, r8, blend, prelu)


def _s1(ktbl_in, qtbl_in, vtbl_in, src_in, dst_in):
    mesh = plsc.VectorSubcoreMesh(core_axis_name="c", subcore_axis_name="s")
    eshape = jax.ShapeDtypeStruct((E_PAD, HC), jnp.float32)
    EW2 = E_PAD // 32

    NCH = EW2 // 256

    @functools.partial(
        pl.kernel,
        out_type=[eshape] * 3,
        mesh=mesh,
        scratch_types=[pltpu.VMEM((4, 128), jnp.int32),
                       pltpu.VMEM((4, 128), jnp.int32),
                       pltpu.VMEM((256, HC), jnp.float32),
                       pltpu.VMEM((256, HC), jnp.float32),
                       pltpu.VMEM((256, HC), jnp.float32),
                       pltpu.SemaphoreType.DMA,
                       pltpu.SemaphoreType.DMA,
                       pltpu.SemaphoreType.DMA],
    )
    def k(ktbl, qtbl, vtbl, src1d, dst1d, ke, qe, ve,
          sidx, didx, kbuf, qbuf, vbuf, isem, gsem, wsem):
        wid = lax.axis_index("s") * 2 + lax.axis_index("c")
        base = wid * EW2

        def chunk(ch, carry):
            eb = base + ch * 256
            icp = []
            for j in range(2):
                icp.append(pltpu.async_copy(
                    src1d.at[pl.ds(eb + j * 128, 128)], sidx.at[j], isem))
                icp.append(pltpu.async_copy(
                    dst1d.at[pl.ds(eb + j * 128, 128)], didx.at[j], isem))
            for cp in icp:
                cp.wait()
            cps = []
            for j in range(2):
                blk = pl.ds(j * 128, 128)
                cps.append(pltpu.async_copy(ktbl.at[sidx.at[j]],
                                            kbuf.at[blk], gsem))
                cps.append(pltpu.async_copy(qtbl.at[didx.at[j]],
                                            qbuf.at[blk], gsem))
                cps.append(pltpu.async_copy(vtbl.at[sidx.at[j]],
                                            vbuf.at[blk], gsem))
            for cp in cps:
                cp.wait()
            ocp = [pltpu.async_copy(kbuf, ke.at[pl.ds(eb, 256)], wsem),
                   pltpu.async_copy(qbuf, qe.at[pl.ds(eb, 256)], wsem),
                   pltpu.async_copy(vbuf, ve.at[pl.ds(eb, 256)], wsem)]
            for cp in ocp:
                cp.wait()
            return carry
        lax.fori_loop(0, EW2 // 256, chunk, 0)

    return k(ktbl_in, qtbl_in, vtbl_in, src_in, dst_in)


def _s3(w_h, exw_h, il_h, d3_h, bounds_h, zacc_h):
    mesh = plsc.VectorSubcoreMesh(core_axis_name="c", subcore_axis_name="s")

    @functools.partial(
        pl.kernel,
        out_type=[jax.ShapeDtypeStruct((2, N_PAD, HC), jnp.float32),
                  jax.ShapeDtypeStruct((2, DEN_ROWS, HC), jnp.float32)],
        mesh=mesh,
        scratch_types=[pltpu.VMEM((1, 128), jnp.int32),
                       pltpu.VMEM((4, 128), jnp.int32),
                       pltpu.VMEM((512, HC), jnp.float32),
                       pltpu.VMEM_SHARED((ACC_ROWS, HC), jnp.float32),
                       pltpu.SemaphoreType.DMA,
                       pltpu.SemaphoreType.DMA],
    )
    def k(w_in, exw_in, il_in, d3_in, bounds, zacc,
          agg_out, den_out,
          bbuf, ibuf, wbuf, acc_sp, gsem, asem):
        c = lax.axis_index("c")
        s = lax.axis_index("s")
        wid = s * 2 + c
        pltpu.sync_copy(bounds, bbuf)
        bv = bbuf[0, pl.ds(0, 16)]

        for p in range(NPASS):
            pltpu.sync_copy(zacc,
                            acc_sp.at[pl.ds(s * (ACC_ROWS // 16), ACC_ROWS // 16)])
            plsc.subcore_barrier()
            b0 = bv[p] // 128
            b1 = (bv[p + 1] + 127) // 128
            nloop = (b1 - b0 + 127) // 128

            def kiter(ki, carry):
                for j in range(4):
                    blk = b0 + wid * 4 + ki * 128 + j

                    @pl.when(blk < b1)
                    def _():
                        cps = [pltpu.async_copy(
                                   w_in.at[pl.ds(blk * 128, 128)],
                                   wbuf.at[pl.ds(j * 128, 128)], gsem),
                               pltpu.async_copy(
                                   il_in.at[p].at[pl.ds(blk * 128, 128)],
                                   ibuf.at[j], gsem)]
                        for cp in cps:
                            cp.wait()
                        pltpu.async_copy(wbuf.at[pl.ds(j * 128, 128)],
                                         acc_sp.at[ibuf.at[j]], asem,
                                         add=True).wait()
                return carry
            lax.fori_loop(0, nloop, kiter, 0)
            plsc.subcore_barrier()

            @pl.when(s == 0)
            def _():
                pltpu.sync_copy(acc_sp.at[pl.ds(0, RANGE)],
                                agg_out.at[c].at[pl.ds(p * RANGE, RANGE)])
            plsc.subcore_barrier()

        pltpu.sync_copy(zacc.at[pl.ds(0, DEN_ROWS // 16)],
                        acc_sp.at[pl.ds(s * (DEN_ROWS // 16), DEN_ROWS // 16)])
        plsc.subcore_barrier()

        def dchunk(ch, carry):
            for j in range(2):
                blk = wid * (NBLK // 32) + ch * 2 + j
                cps = [pltpu.async_copy(exw_in.at[pl.ds(blk * 128, 128)],
                                        wbuf.at[pl.ds(j * 128, 128)], gsem),
                       pltpu.async_copy(d3_in.at[pl.ds(blk * 128, 128)],
                                        ibuf.at[j], gsem)]
                for cp in cps:
                    cp.wait()
                pltpu.async_copy(wbuf.at[pl.ds(j * 128, 128)],
                                 acc_sp.at[ibuf.at[j]], asem, add=True).wait()
            return carry
        lax.fori_loop(0, NBLK // 32 // 2, dchunk, 0)
        plsc.subcore_barrier()

        @pl.when(s == 0)
        def _():
            pltpu.sync_copy(acc_sp.at[pl.ds(0, DEN_ROWS)], den_out.at[c])

    return k(w_h, exw_h, il_h, d3_h, bounds_h, zacc_h)


def _edge_prep(edge):
    padi = N_NODE + jnp.arange(E_PAD - E, dtype=jnp.int32) % (N_PAD - N_NODE)
    src = jnp.concatenate([edge[0], padi])
    dst = jnp.concatenate([edge[1], padi])
    key = dst // RANGE
    _, src, dst = lax.sort((key, src, dst), num_keys=1)
    counts = jnp.bincount(dst // RANGE, length=NPASS)
    starts = jnp.concatenate([jnp.zeros((1,), counts.dtype),
                              jnp.cumsum(counts)]).astype(jnp.int32)
    bounds = jnp.zeros((1, 128), jnp.int32).at[0, :NPASS + 1].set(starts)
    il = jnp.stack([
        jnp.where((dst >= p * RANGE) & (dst < (p + 1) * RANGE),
                  dst - p * RANGE, RANGE).astype(jnp.int32)
        for p in range(NPASS)])
    d3 = (dst // 8).astype(jnp.int32)
    d8 = (dst % 8).astype(jnp.int32).reshape(E_PAD, 1)
    return src, dst, il, d3, d8, bounds


def kernel(x_paper, x_author, edge_index_p2a, edge_index_a2p,
           W_in_p, b_in_p, W_in_a, b_in_a,
           Wk_p, bk_p, Wq_p, bq_p, Wv_p, bv_p,
           Wk_a, bk_a, Wq_a, bq_a, Wv_a, bv_a,
           a_p2a, m_p2a, p_p2a, a_a2p, m_a2p, p_a2p,
           Wo_p, bo_p, Wo_a, bo_a, skip_p, skip_a, prelu_w):
    f32 = jnp.float32

    A_p2a = _block_diag(a_p2a); M_p2a = _block_diag(m_p2a)
    A_a2p = _block_diag(a_a2p); M_a2p = _block_diag(m_a2p)
    s_p2a = jnp.repeat(p_p2a, DH) / np.sqrt(DH)
    s_a2p = jnp.repeat(p_a2p, DH) / np.sqrt(DH)

    def pad_rows(x):
        return jnp.concatenate([x, jnp.zeros((N_PAD - N_NODE, HC), f32)], 0)

    b2 = lambda b: b.reshape(1, HC)
    xn_p, q_p, k_p, v_p = _proj(
        pad_rows(x_paper), W_in_p.T, b2(b_in_p),
        Wq_p.T * s_a2p[None, :], b2(bq_p * s_a2p),
        Wk_p.T @ A_p2a, b2(bk_p @ A_p2a),
        Wv_p.T @ M_p2a, b2(bv_p @ M_p2a))
    xn_a, q_a, k_a, v_a = _proj(
        pad_rows(x_author), W_in_a.T, b2(b_in_a),
        Wq_a.T * s_p2a[None, :], b2(bq_a * s_p2a),
        Wk_a.T @ A_a2p, b2(bk_a @ A_a2p),
        Wv_a.T @ M_a2p, b2(bv_a @ M_a2p))

    sp, dp, il_p, d3_p, d8_p, bd_p = _edge_prep(edge_index_p2a)
    sa, da, il_a, d3_a, d8_a, bd_a = _edge_prep(edge_index_a2p)

    s16 = jnp.zeros((HC, DH), f32)
    r8 = jnp.zeros((DH, HC), f32)
    for h in range(HEADS):
        s16 = s16.at[h * DH:(h + 1) * DH, h].set(1.0)
        r8 = r8.at[h, h * DH:(h + 1) * DH].set(1.0)
    zacc = jnp.zeros((ACC_ROWS // 16, HC), f32)

    ke_p, qe_p, ve_p = _s1(k_p, q_a, v_p, sp, dp)
    w_p2a, exw_p2a = _s2(ke_p, qe_p, ve_p, d8_p, s16, r8)
    ke_a, qe_a, ve_a = _s1(k_a, q_p, v_a, sa, da)
    w_a2p, exw_a2p = _s2(ke_a, qe_a, ve_a, d8_a, s16, r8)

    agg_a2, den_a2 = _s3(w_p2a, exw_p2a, il_p, d3_p, bd_p, zacc)
    agg_p2, den_p2 = _s3(w_a2p, exw_a2p, il_a, d3_a, bd_a, zacc)

    bp = jax.nn.sigmoid(skip_p).reshape(1, 1)
    ba = jax.nn.sigmoid(skip_a).reshape(1, 1)
    out_p = _post(agg_p2[0], agg_p2[1],
                  den_p2[0].reshape(N_PAD, DH), den_p2[1].reshape(N_PAD, DH),
                  xn_p, Wo_p.T, b2(bo_p), r8, bp, b2(prelu_w))
    out_a = _post(agg_a2[0], agg_a2[1],
                  den_a2[0].reshape(N_PAD, DH), den_a2[1].reshape(N_PAD, DH),
                  xn_a, Wo_a.T, b2(bo_a), r8, ba, b2(prelu_w))
    return out_p[:N_NODE], out_a[:N_NODE]


# speedup vs baseline: 1.4319x; 1.0063x over previous
import functools

import jax
import jax.numpy as jnp
import numpy as np
from jax import lax
from jax.experimental import pallas as pl
from jax.experimental.pallas import tpu as pltpu
from jax.experimental.pallas import tpu_sc as plsc

N_NODE = 50000
E = 300000
HC = 128
HEADS = 8
DH = 16

N_PAD = 50176
E_PAD = 303104
EW = E_PAD // 16
NBLK = E_PAD // 128
NPASS = 8
RANGE = N_PAD // NPASS
ACC_ROWS = RANGE + 16
DEN_ROWS = N_PAD // 8


def _block_diag(a):
    out = jnp.zeros((HC, HC), jnp.float32)
    for h in range(HEADS):
        out = out.at[h * DH:(h + 1) * DH, h * DH:(h + 1) * DH].set(a[h])
    return out


def _proj_body(x_ref, wi_ref, bi_ref, wq_ref, bq_ref, wk_ref, bk_ref,
               wv_ref, bv_ref, xn_ref, q_ref, k_ref, v_ref):
    xn = jnp.dot(x_ref[...], wi_ref[...], preferred_element_type=jnp.float32)
    xn = xn + bi_ref[...]
    xn_ref[...] = xn
    q_ref[...] = jnp.dot(xn, wq_ref[...], preferred_element_type=jnp.float32) + bq_ref[...]
    k_ref[...] = jnp.dot(xn, wk_ref[...], preferred_element_type=jnp.float32) + bk_ref[...]
    v_ref[...] = jnp.dot(xn, wv_ref[...], preferred_element_type=jnp.float32) + bv_ref[...]


def _proj(x, wi, bi, wq, bq, wk, bk, wv, bv):
    row = pl.BlockSpec((512, HC), lambda i: (i, 0))
    wsp = pl.BlockSpec((HC, HC), lambda i: (0, 0))
    bsp = pl.BlockSpec((1, HC), lambda i: (0, 0))
    out = jax.ShapeDtypeStruct((N_PAD, HC), jnp.float32)
    return pl.pallas_call(
        _proj_body,
        grid=(N_PAD // 512,),
        in_specs=[row, wsp, bsp, wsp, bsp, wsp, bsp, wsp, bsp],
        out_specs=[row, row, row, row],
        out_shape=[out, out, out, out],
    )(x, wi, bi, wq, bq, wk, bk, wv, bv)


def _s2_body(k_ref, q_ref, v_ref, d8_ref, s16_ref, r8_ref, w_ref, exw_ref):
    prod = k_ref[...] * q_ref[...]
    ex16 = jnp.exp(jnp.dot(prod, s16_ref[...],
                           preferred_element_type=jnp.float32))
    ex_t = jnp.dot(ex16, r8_ref[...], preferred_element_type=jnp.float32)
    w_ref[...] = v_ref[...] * ex_t
    colg = lax.broadcasted_iota(jnp.int32, (512, HC), 1) // DH
    exw_ref[...] = jnp.where(colg == d8_ref[...], ex_t, 0.0)


def _s2(ke, qe, ve, d8, s16, r8):
    row = pl.BlockSpec((512, HC), lambda i: (i, 0))
    return pl.pallas_call(
        _s2_body,
        grid=(E_PAD // 512,),
        in_specs=[row, row, row,
                  pl.BlockSpec((512, 1), lambda i: (i, 0)),
                  pl.BlockSpec((HC, DH), lambda i: (0, 0)),
                  pl.BlockSpec((DH, HC), lambda i: (0, 0))],
        out_specs=[row, row],
        out_shape=[jax.ShapeDtypeStruct((E_PAD, HC), jnp.float32),
                   jax.ShapeDtypeStruct((E_PAD, HC), jnp.float32)],
    )(ke, qe, ve, d8, s16, r8)


def _post_body(agg0_ref, agg1_ref, den0_ref, den1_ref, xn_ref, wo_ref,
               bo_ref, r8_ref, blend_ref, prelu_ref, o_ref):
    den = den0_ref[...] + den1_ref[...]
    dw = jnp.dot(den, r8_ref[...], preferred_element_type=jnp.float32)
    a = (agg0_ref[...] + agg1_ref[...]) / (dw + 1e-16)
    g = jax.nn.gelu(a)
    o = jnp.dot(g, wo_ref[...], preferred_element_type=jnp.float32) + bo_ref[...]
    b = blend_ref[0, 0]
    o = b * o + (1.0 - b) * xn_ref[...]
    o_ref[...] = jnp.where(o > 0, o, prelu_ref[...] * o)


def _post(agg0, agg1, den0, den1, xn, wo, bo, r8, blend, prelu):
    row = pl.BlockSpec((512, HC), lambda i: (i, 0))
    wsp = pl.BlockSpec((HC, HC), lambda i: (0, 0))
    bsp = pl.BlockSpec((1, HC), lambda i: (0, 0))
    dsp = pl.BlockSpec((512, DH), lambda i: (i, 0))
    return pl.pallas_call(
        _post_body,
        grid=(N_PAD // 512,),
        in_specs=[row, row, dsp, dsp, row, wsp, bsp,
                  pl.BlockSpec((DH, HC), lambda i: (0, 0)),
                  pl.BlockSpec((1, 1), lambda i: (0, 0)),
                  bsp],
        out_specs=row,
        out_shape=jax.ShapeDtypeStruct((N_PAD, HC), jnp.float32),
    )(agg0, agg1, den0, den1, xn, wo, bo, r8, blend, prelu)


def _s1(ktbl_in, qtbl_in, vtbl_in, src_in, dst_in):
    mesh = plsc.VectorSubcoreMesh(core_axis_name="c", subcore_axis_name="s")
    eshape = jax.ShapeDtypeStruct((E_PAD, HC), jnp.float32)
    EW2 = E_PAD // 32

    NCH = EW2 // 256

    @functools.partial(
        pl.kernel,
        out_type=[eshape] * 3,
        mesh=mesh,
        scratch_types=[pltpu.VMEM((4, 128), jnp.int32),
                       pltpu.VMEM((4, 128), jnp.int32),
                       pltpu.VMEM((256, HC), jnp.float32),
                       pltpu.VMEM((256, HC), jnp.float32),
                       pltpu.VMEM((256, HC), jnp.float32),
                       pltpu.SemaphoreType.DMA,
                       pltpu.SemaphoreType.DMA,
                       pltpu.SemaphoreType.DMA],
    )
    def k(ktbl, qtbl, vtbl, src1d, dst1d, ke, qe, ve,
          sidx, didx, kbuf, qbuf, vbuf, isem, gsem, wsem):
        wid = lax.axis_index("s") * 2 + lax.axis_index("c")
        base = wid * EW2

        def chunk(ch, carry):
            eb = base + ch * 256
            icp = []
            for j in range(2):
                icp.append(pltpu.async_copy(
                    src1d.at[pl.ds(eb + j * 128, 128)], sidx.at[j], isem))
                icp.append(pltpu.async_copy(
                    dst1d.at[pl.ds(eb + j * 128, 128)], didx.at[j], isem))
            for cp in icp:
                cp.wait()
            cps = []
            for j in range(2):
                blk = pl.ds(j * 128, 128)
                cps.append(pltpu.async_copy(ktbl.at[sidx.at[j]],
                                            kbuf.at[blk], gsem))
                cps.append(pltpu.async_copy(qtbl.at[didx.at[j]],
                                            qbuf.at[blk], gsem))
                cps.append(pltpu.async_copy(vtbl.at[sidx.at[j]],
                                            vbuf.at[blk], gsem))
            for cp in cps:
                cp.wait()
            ocp = [pltpu.async_copy(kbuf, ke.at[pl.ds(eb, 256)], wsem),
                   pltpu.async_copy(qbuf, qe.at[pl.ds(eb, 256)], wsem),
                   pltpu.async_copy(vbuf, ve.at[pl.ds(eb, 256)], wsem)]
            for cp in ocp:
                cp.wait()
            return carry
        lax.fori_loop(0, EW2 // 256, chunk, 0)

    return k(ktbl_in, qtbl_in, vtbl_in, src_in, dst_in)


def _s3(w_h, exw_h, il_h, d3_h, bounds_h, zacc_h):
    mesh = plsc.VectorSubcoreMesh(core_axis_name="c", subcore_axis_name="s")

    @functools.partial(
        pl.kernel,
        out_type=[jax.ShapeDtypeStruct((2, N_PAD, HC), jnp.float32),
                  jax.ShapeDtypeStruct((2, DEN_ROWS, HC), jnp.float32)],
        mesh=mesh,
        scratch_types=[pltpu.VMEM((1, 128), jnp.int32),
                       pltpu.VMEM((4, 128), jnp.int32),
                       pltpu.VMEM((512, HC), jnp.float32),
                       pltpu.VMEM_SHARED((ACC_ROWS, HC), jnp.float32),
                       pltpu.SemaphoreType.DMA,
                       pltpu.SemaphoreType.DMA],
    )
    def k(w_in, exw_in, il_in, d3_in, bounds, zacc,
          agg_out, den_out,
          bbuf, ibuf, wbuf, acc_sp, gsem, asem):
        c = lax.axis_index("c")
        s = lax.axis_index("s")
        wid = s * 2 + c
        pltpu.sync_copy(bounds, bbuf)
        bv = bbuf[0, pl.ds(0, 16)]

        for p in range(NPASS):
            pltpu.sync_copy(zacc,
                            acc_sp.at[pl.ds(s * (ACC_ROWS // 16), ACC_ROWS // 16)])
            plsc.subcore_barrier()
            b0 = bv[p] // 128
            b1 = (bv[p + 1] + 127) // 128
            nloop = (b1 - b0 + 127) // 128

            def kiter(ki, carry):
                for j in range(4):
                    blk = b0 + wid * 4 + ki * 128 + j

                    @pl.when(blk < b1)
                    def _(blk=blk, j=j):
                        cps = [pltpu.async_copy(
                                   w_in.at[pl.ds(blk * 128, 128)],
                                   wbuf.at[pl.ds(j * 128, 128)], gsem),
                               pltpu.async_copy(
                                   il_in.at[p].at[pl.ds(blk * 128, 128)],
                                   ibuf.at[j], gsem)]
                        for cp in cps:
                            cp.wait()
                        pltpu.async_copy(wbuf.at[pl.ds(j * 128, 128)],
                                         acc_sp.at[ibuf.at[j]], asem,
                                         add=True).wait()
                return carry
            lax.fori_loop(0, nloop, kiter, 0)
            plsc.subcore_barrier()

            @pl.when(s == 0)
            def _():
                pltpu.sync_copy(acc_sp.at[pl.ds(0, RANGE)],
                                agg_out.at[c].at[pl.ds(p * RANGE, RANGE)])
            plsc.subcore_barrier()

        pltpu.sync_copy(zacc.at[pl.ds(0, DEN_ROWS // 16)],
                        acc_sp.at[pl.ds(s * (DEN_ROWS // 16), DEN_ROWS // 16)])
        plsc.subcore_barrier()

        def dchunk(ch, carry):
            cps = []
            for j in range(2):
                blk = wid * (NBLK // 32) + ch * 2 + j
                cps.append(pltpu.async_copy(exw_in.at[pl.ds(blk * 128, 128)],
                                            wbuf.at[pl.ds(j * 128, 128)], gsem))
                cps.append(pltpu.async_copy(d3_in.at[pl.ds(blk * 128, 128)],
                                            ibuf.at[j], gsem))
            for cp in cps:
                cp.wait()
            scs = [pltpu.async_copy(wbuf.at[pl.ds(j * 128, 128)],
                                    acc_sp.at[ibuf.at[j]], asem, add=True)
                   for j in range(2)]
            for cp in scs:
                cp.wait()
            return carry
        lax.fori_loop(0, NBLK // 32 // 2, dchunk, 0)
        plsc.subcore_barrier()

        @pl.when(s == 0)
        def _():
            pltpu.sync_copy(acc_sp.at[pl.ds(0, DEN_ROWS)], den_out.at[c])

    return k(w_h, exw_h, il_h, d3_h, bounds_h, zacc_h)


def _edge_prep(edge):
    padi = N_NODE + jnp.arange(E_PAD - E, dtype=jnp.int32) % (N_PAD - N_NODE)
    src = jnp.concatenate([edge[0], padi])
    dst = jnp.concatenate([edge[1], padi])
    key = dst // RANGE
    _, src, dst = lax.sort((key, src, dst), num_keys=1)
    counts = jnp.bincount(dst // RANGE, length=NPASS)
    starts = jnp.concatenate([jnp.zeros((1,), counts.dtype),
                              jnp.cumsum(counts)]).astype(jnp.int32)
    bounds = jnp.zeros((1, 128), jnp.int32).at[0, :NPASS + 1].set(starts)
    il = jnp.stack([
        jnp.where((dst >= p * RANGE) & (dst < (p + 1) * RANGE),
                  dst - p * RANGE, RANGE).astype(jnp.int32)
        for p in range(NPASS)])
    d3 = (dst // 8).astype(jnp.int32)
    d8 = (dst % 8).astype(jnp.int32).reshape(E_PAD, 1)
    return src, dst, il, d3, d8, bounds


def kernel(x_paper, x_author, edge_index_p2a, edge_index_a2p,
           W_in_p, b_in_p, W_in_a, b_in_a,
           Wk_p, bk_p, Wq_p, bq_p, Wv_p, bv_p,
           Wk_a, bk_a, Wq_a, bq_a, Wv_a, bv_a,
           a_p2a, m_p2a, p_p2a, a_a2p, m_a2p, p_a2p,
           Wo_p, bo_p, Wo_a, bo_a, skip_p, skip_a, prelu_w):
    f32 = jnp.float32

    A_p2a = _block_diag(a_p2a); M_p2a = _block_diag(m_p2a)
    A_a2p = _block_diag(a_a2p); M_a2p = _block_diag(m_a2p)
    s_p2a = jnp.repeat(p_p2a, DH) / np.sqrt(DH)
    s_a2p = jnp.repeat(p_a2p, DH) / np.sqrt(DH)

    def pad_rows(x):
        return jnp.concatenate([x, jnp.zeros((N_PAD - N_NODE, HC), f32)], 0)

    b2 = lambda b: b.reshape(1, HC)
    xn_p, q_p, k_p, v_p = _proj(
        pad_rows(x_paper), W_in_p.T, b2(b_in_p),
        Wq_p.T * s_a2p[None, :], b2(bq_p * s_a2p),
        Wk_p.T @ A_p2a, b2(bk_p @ A_p2a),
        Wv_p.T @ M_p2a, b2(bv_p @ M_p2a))
    xn_a, q_a, k_a, v_a = _proj(
        pad_rows(x_author), W_in_a.T, b2(b_in_a),
        Wq_a.T * s_p2a[None, :], b2(bq_a * s_p2a),
        Wk_a.T @ A_a2p, b2(bk_a @ A_a2p),
        Wv_a.T @ M_a2p, b2(bv_a @ M_a2p))

    sp, dp, il_p, d3_p, d8_p, bd_p = _edge_prep(edge_index_p2a)
    sa, da, il_a, d3_a, d8_a, bd_a = _edge_prep(edge_index_a2p)

    s16 = jnp.zeros((HC, DH), f32)
    r8 = jnp.zeros((DH, HC), f32)
    for h in range(HEADS):
        s16 = s16.at[h * DH:(h + 1) * DH, h].set(1.0)
        r8 = r8.at[h, h * DH:(h + 1) * DH].set(1.0)
    zacc = jnp.zeros((ACC_ROWS // 16, HC), f32)

    ke_p, qe_p, ve_p = _s1(k_p, q_a, v_p, sp, dp)
    w_p2a, exw_p2a = _s2(ke_p, qe_p, ve_p, d8_p, s16, r8)
    ke_a, qe_a, ve_a = _s1(k_a, q_p, v_a, sa, da)
    w_a2p, exw_a2p = _s2(ke_a, qe_a, ve_a, d8_a, s16, r8)

    agg_a2, den_a2 = _s3(w_p2a, exw_p2a, il_p, d3_p, bd_p, zacc)
    agg_p2, den_p2 = _s3(w_a2p, exw_a2p, il_a, d3_a, bd_a, zacc)

    bp = jax.nn.sigmoid(skip_p).reshape(1, 1)
    ba = jax.nn.sigmoid(skip_a).reshape(1, 1)
    out_p = _post(agg_p2[0], agg_p2[1],
                  den_p2[0].reshape(N_PAD, DH), den_p2[1].reshape(N_PAD, DH),
                  xn_p, Wo_p.T, b2(bo_p), r8, bp, b2(prelu_w))
    out_a = _post(agg_a2[0], agg_a2[1],
                  den_a2[0].reshape(N_PAD, DH), den_a2[1].reshape(N_PAD, DH),
                  xn_a, Wo_a.T, b2(bo_a), r8, ba, b2(prelu_w))
    return out_p[:N_NODE], out_a[:N_NODE]


# speedup vs baseline: 1.4493x; 1.0122x over previous
import functools

import jax
import jax.numpy as jnp
import numpy as np
from jax import lax
from jax.experimental import pallas as pl
from jax.experimental.pallas import tpu as pltpu
from jax.experimental.pallas import tpu_sc as plsc

N_NODE = 50000
E = 300000
HC = 128
HEADS = 8
DH = 16

N_PAD = 50176
E_PAD = 303104
EW = E_PAD // 16
NBLK = E_PAD // 128
NPASS = 8
RANGE = N_PAD // NPASS
ACC_ROWS = RANGE + 16
DEN_ROWS = N_PAD // 8


def _block_diag(a):
    out = jnp.zeros((HC, HC), jnp.float32)
    for h in range(HEADS):
        out = out.at[h * DH:(h + 1) * DH, h * DH:(h + 1) * DH].set(a[h])
    return out


def _proj_body(x_ref, wi_ref, bi_ref, wq_ref, bq_ref, wk_ref, bk_ref,
               wv_ref, bv_ref, xn_ref, q_ref, k_ref, v_ref):
    xn = jnp.dot(x_ref[...], wi_ref[...], preferred_element_type=jnp.float32)
    xn = xn + bi_ref[...]
    xn_ref[...] = xn
    q_ref[...] = jnp.dot(xn, wq_ref[...], preferred_element_type=jnp.float32) + bq_ref[...]
    k_ref[...] = jnp.dot(xn, wk_ref[...], preferred_element_type=jnp.float32) + bk_ref[...]
    v_ref[...] = jnp.dot(xn, wv_ref[...], preferred_element_type=jnp.float32) + bv_ref[...]


def _proj(x, wi, bi, wq, bq, wk, bk, wv, bv):
    row = pl.BlockSpec((512, HC), lambda i: (i, 0))
    wsp = pl.BlockSpec((HC, HC), lambda i: (0, 0))
    bsp = pl.BlockSpec((1, HC), lambda i: (0, 0))
    out = jax.ShapeDtypeStruct((N_PAD, HC), jnp.float32)
    return pl.pallas_call(
        _proj_body,
        grid=(N_PAD // 512,),
        in_specs=[row, wsp, bsp, wsp, bsp, wsp, bsp, wsp, bsp],
        out_specs=[row, row, row, row],
        out_shape=[out, out, out, out],
    )(x, wi, bi, wq, bq, wk, bk, wv, bv)


def _s2_body(k_ref, q_ref, v_ref, d8_ref, s16_ref, r8_ref, w_ref, exw_ref):
    prod = k_ref[...] * q_ref[...]
    ex16 = jnp.exp(jnp.dot(prod, s16_ref[...],
                           preferred_element_type=jnp.float32))
    ex_t = jnp.dot(ex16, r8_ref[...], preferred_element_type=jnp.float32)
    w_ref[...] = v_ref[...] * ex_t
    colg = lax.broadcasted_iota(jnp.int32, (512, HC), 1) // DH
    exw_ref[...] = jnp.where(colg == d8_ref[...], ex_t, 0.0)


def _s2(ke, qe, ve, d8, s16, r8):
    row = pl.BlockSpec((512, HC), lambda i: (i, 0))
    return pl.pallas_call(
        _s2_body,
        grid=(E_PAD // 512,),
        in_specs=[row, row, row,
                  pl.BlockSpec((512, 1), lambda i: (i, 0)),
                  pl.BlockSpec((HC, DH), lambda i: (0, 0)),
                  pl.BlockSpec((DH, HC), lambda i: (0, 0))],
        out_specs=[row, row],
        out_shape=[jax.ShapeDtypeStruct((E_PAD, HC), jnp.float32),
                   jax.ShapeDtypeStruct((E_PAD, HC), jnp.float32)],
    )(ke, qe, ve, d8, s16, r8)


def _post_body(agg0_ref, agg1_ref, den0_ref, den1_ref, xn_ref, wo_ref,
               bo_ref, r8_ref, blend_ref, prelu_ref, o_ref):
    den = den0_ref[...] + den1_ref[...]
    dw = jnp.dot(den, r8_ref[...], preferred_element_type=jnp.float32)
    a = (agg0_ref[...] + agg1_ref[...]) / (dw + 1e-16)
    g = jax.nn.gelu(a)
    o = jnp.dot(g, wo_ref[...], preferred_element_type=jnp.float32) + bo_ref[...]
    b = blend_ref[0, 0]
    o = b * o + (1.0 - b) * xn_ref[...]
    o_ref[...] = jnp.where(o > 0, o, prelu_ref[...] * o)


def _post(agg0, agg1, den0, den1, xn, wo, bo, r8, blend, prelu):
    row = pl.BlockSpec((512, HC), lambda i: (i, 0))
    wsp = pl.BlockSpec((HC, HC), lambda i: (0, 0))
    bsp = pl.BlockSpec((1, HC), lambda i: (0, 0))
    dsp = pl.BlockSpec((512, DH), lambda i: (i, 0))
    return pl.pallas_call(
        _post_body,
        grid=(N_PAD // 512,),
        in_specs=[row, row, dsp, dsp, row, wsp, bsp,
                  pl.BlockSpec((DH, HC), lambda i: (0, 0)),
                  pl.BlockSpec((1, 1), lambda i: (0, 0)),
                  bsp],
        out_specs=row,
        out_shape=jax.ShapeDtypeStruct((N_PAD, HC), jnp.float32),
    )(agg0, agg1, den0, den1, xn, wo, bo, r8, blend, prelu)


def _s1(ktbl_in, qtbl_in, vtbl_in, src_in, dst_in):
    mesh = plsc.VectorSubcoreMesh(core_axis_name="c", subcore_axis_name="s")
    eshape = jax.ShapeDtypeStruct((E_PAD, HC), jnp.float32)
    EW2 = E_PAD // 32

    NCH = EW2 // 256

    @functools.partial(
        pl.kernel,
        out_type=[eshape] * 3,
        mesh=mesh,
        scratch_types=[pltpu.VMEM((4, 128), jnp.int32),
                       pltpu.VMEM((4, 128), jnp.int32),
                       pltpu.VMEM((256, HC), jnp.float32),
                       pltpu.VMEM((256, HC), jnp.float32),
                       pltpu.VMEM((256, HC), jnp.float32),
                       pltpu.SemaphoreType.DMA,
                       pltpu.SemaphoreType.DMA,
                       pltpu.SemaphoreType.DMA],
    )
    def k(ktbl, qtbl, vtbl, src1d, dst1d, ke, qe, ve,
          sidx, didx, kbuf, qbuf, vbuf, isem, gsem, wsem):
        wid = lax.axis_index("s") * 2 + lax.axis_index("c")
        base = wid * EW2

        def chunk(ch, carry):
            eb = base + ch * 256
            icp = []
            for j in range(2):
                icp.append(pltpu.async_copy(
                    src1d.at[pl.ds(eb + j * 128, 128)], sidx.at[j], isem))
                icp.append(pltpu.async_copy(
                    dst1d.at[pl.ds(eb + j * 128, 128)], didx.at[j], isem))
            for cp in icp:
                cp.wait()
            cps = []
            for j in range(2):
                blk = pl.ds(j * 128, 128)
                cps.append(pltpu.async_copy(ktbl.at[sidx.at[j]],
                                            kbuf.at[blk], gsem))
                cps.append(pltpu.async_copy(qtbl.at[didx.at[j]],
                                            qbuf.at[blk], gsem))
                cps.append(pltpu.async_copy(vtbl.at[sidx.at[j]],
                                            vbuf.at[blk], gsem))
            for cp in cps:
                cp.wait()
            ocp = [pltpu.async_copy(kbuf, ke.at[pl.ds(eb, 256)], wsem),
                   pltpu.async_copy(qbuf, qe.at[pl.ds(eb, 256)], wsem),
                   pltpu.async_copy(vbuf, ve.at[pl.ds(eb, 256)], wsem)]
            for cp in ocp:
                cp.wait()
            return carry
        lax.fori_loop(0, EW2 // 256, chunk, 0)

    return k(ktbl_in, qtbl_in, vtbl_in, src_in, dst_in)


def _s3(w_h, exw_h, il_h, d3_h, bounds_h, zacc_h):
    mesh = plsc.VectorSubcoreMesh(core_axis_name="c", subcore_axis_name="s")

    @functools.partial(
        pl.kernel,
        out_type=[jax.ShapeDtypeStruct((2, N_PAD, HC), jnp.float32),
                  jax.ShapeDtypeStruct((2, DEN_ROWS, HC), jnp.float32)],
        mesh=mesh,
        scratch_types=[pltpu.VMEM((1, 128), jnp.int32),
                       pltpu.VMEM((4, 128), jnp.int32),
                       pltpu.VMEM((512, HC), jnp.float32),
                       pltpu.VMEM_SHARED((ACC_ROWS, HC), jnp.float32),
                       pltpu.SemaphoreType.DMA,
                       pltpu.SemaphoreType.DMA],
    )
    def k(w_in, exw_in, il_in, d3_in, bounds, zacc,
          agg_out, den_out,
          bbuf, ibuf, wbuf, acc_sp, gsem, asem):
        c = lax.axis_index("c")
        s = lax.axis_index("s")
        wid = s * 2 + c
        pltpu.sync_copy(bounds, bbuf)
        bv = bbuf[0, pl.ds(0, 16)]

        for p in range(NPASS):
            pltpu.sync_copy(zacc,
                            acc_sp.at[pl.ds(s * (ACC_ROWS // 16), ACC_ROWS // 16)])
            plsc.subcore_barrier()
            b0 = bv[p] // 128
            b1 = (bv[p + 1] + 127) // 128
            nloop = (b1 - b0 + 127) // 128

            nfull = jnp.clip((b1 - b0 - wid * 4 - 4) // 128 + 1, 0, nloop)

            def kfull(ki, carry):
                cps = []
                for j in range(4):
                    blk = b0 + wid * 4 + ki * 128 + j
                    cps.append(pltpu.async_copy(
                        w_in.at[pl.ds(blk * 128, 128)],
                        wbuf.at[pl.ds(j * 128, 128)], gsem))
                    cps.append(pltpu.async_copy(
                        il_in.at[p].at[pl.ds(blk * 128, 128)],
                        ibuf.at[j], gsem))
                for cp in cps:
                    cp.wait()
                scs = [pltpu.async_copy(wbuf.at[pl.ds(j * 128, 128)],
                                        acc_sp.at[ibuf.at[j]], asem, add=True)
                       for j in range(4)]
                for cp in scs:
                    cp.wait()
                return carry
            lax.fori_loop(0, nfull, kfull, 0)

            def kiter(ki, carry):
                for j in range(4):
                    blk = b0 + wid * 4 + ki * 128 + j

                    @pl.when(blk < b1)
                    def _(blk=blk, j=j):
                        cps = [pltpu.async_copy(
                                   w_in.at[pl.ds(blk * 128, 128)],
                                   wbuf.at[pl.ds(j * 128, 128)], gsem),
                               pltpu.async_copy(
                                   il_in.at[p].at[pl.ds(blk * 128, 128)],
                                   ibuf.at[j], gsem)]
                        for cp in cps:
                            cp.wait()
                        pltpu.async_copy(wbuf.at[pl.ds(j * 128, 128)],
                                         acc_sp.at[ibuf.at[j]], asem,
                                         add=True).wait()
                return carry
            lax.fori_loop(nfull, nloop, kiter, 0)
            plsc.subcore_barrier()

            @pl.when(s == 0)
            def _():
                pltpu.sync_copy(acc_sp.at[pl.ds(0, RANGE)],
                                agg_out.at[c].at[pl.ds(p * RANGE, RANGE)])
            plsc.subcore_barrier()

        pltpu.sync_copy(zacc.at[pl.ds(0, DEN_ROWS // 16)],
                        acc_sp.at[pl.ds(s * (DEN_ROWS // 16), DEN_ROWS // 16)])
        plsc.subcore_barrier()

        def dchunk(ch, carry):
            cps = []
            for j in range(2):
                blk = wid * (NBLK // 32) + ch * 2 + j
                cps.append(pltpu.async_copy(exw_in.at[pl.ds(blk * 128, 128)],
                                            wbuf.at[pl.ds(j * 128, 128)], gsem))
                cps.append(pltpu.async_copy(d3_in.at[pl.ds(blk * 128, 128)],
                                            ibuf.at[j], gsem))
            for cp in cps:
                cp.wait()
            scs = [pltpu.async_copy(wbuf.at[pl.ds(j * 128, 128)],
                                    acc_sp.at[ibuf.at[j]], asem, add=True)
                   for j in range(2)]
            for cp in scs:
                cp.wait()
            return carry
        lax.fori_loop(0, NBLK // 32 // 2, dchunk, 0)
        plsc.subcore_barrier()

        @pl.when(s == 0)
        def _():
            pltpu.sync_copy(acc_sp.at[pl.ds(0, DEN_ROWS)], den_out.at[c])

    return k(w_h, exw_h, il_h, d3_h, bounds_h, zacc_h)


def _edge_prep(edge):
    padi = N_NODE + jnp.arange(E_PAD - E, dtype=jnp.int32) % (N_PAD - N_NODE)
    src = jnp.concatenate([edge[0], padi])
    dst = jnp.concatenate([edge[1], padi])
    key = dst // RANGE
    _, src, dst = lax.sort((key, src, dst), num_keys=1)
    counts = jnp.bincount(dst // RANGE, length=NPASS)
    starts = jnp.concatenate([jnp.zeros((1,), counts.dtype),
                              jnp.cumsum(counts)]).astype(jnp.int32)
    bounds = jnp.zeros((1, 128), jnp.int32).at[0, :NPASS + 1].set(starts)
    il = jnp.stack([
        jnp.where((dst >= p * RANGE) & (dst < (p + 1) * RANGE),
                  dst - p * RANGE, RANGE).astype(jnp.int32)
        for p in range(NPASS)])
    d3 = (dst // 8).astype(jnp.int32)
    d8 = (dst % 8).astype(jnp.int32).reshape(E_PAD, 1)
    return src, dst, il, d3, d8, bounds


def kernel(x_paper, x_author, edge_index_p2a, edge_index_a2p,
           W_in_p, b_in_p, W_in_a, b_in_a,
           Wk_p, bk_p, Wq_p, bq_p, Wv_p, bv_p,
           Wk_a, bk_a, Wq_a, bq_a, Wv_a, bv_a,
           a_p2a, m_p2a, p_p2a, a_a2p, m_a2p, p_a2p,
           Wo_p, bo_p, Wo_a, bo_a, skip_p, skip_a, prelu_w):
    f32 = jnp.float32

    A_p2a = _block_diag(a_p2a); M_p2a = _block_diag(m_p2a)
    A_a2p = _block_diag(a_a2p); M_a2p = _block_diag(m_a2p)
    s_p2a = jnp.repeat(p_p2a, DH) / np.sqrt(DH)
    s_a2p = jnp.repeat(p_a2p, DH) / np.sqrt(DH)

    def pad_rows(x):
        return jnp.concatenate([x, jnp.zeros((N_PAD - N_NODE, HC), f32)], 0)

    b2 = lambda b: b.reshape(1, HC)
    xn_p, q_p, k_p, v_p = _proj(
        pad_rows(x_paper), W_in_p.T, b2(b_in_p),
        Wq_p.T * s_a2p[None, :], b2(bq_p * s_a2p),
        Wk_p.T @ A_p2a, b2(bk_p @ A_p2a),
        Wv_p.T @ M_p2a, b2(bv_p @ M_p2a))
    xn_a, q_a, k_a, v_a = _proj(
        pad_rows(x_author), W_in_a.T, b2(b_in_a),
        Wq_a.T * s_p2a[None, :], b2(bq_a * s_p2a),
        Wk_a.T @ A_a2p, b2(bk_a @ A_a2p),
        Wv_a.T @ M_a2p, b2(bv_a @ M_a2p))

    sp, dp, il_p, d3_p, d8_p, bd_p = _edge_prep(edge_index_p2a)
    sa, da, il_a, d3_a, d8_a, bd_a = _edge_prep(edge_index_a2p)

    s16 = jnp.zeros((HC, DH), f32)
    r8 = jnp.zeros((DH, HC), f32)
    for h in range(HEADS):
        s16 = s16.at[h * DH:(h + 1) * DH, h].set(1.0)
        r8 = r8.at[h, h * DH:(h + 1) * DH].set(1.0)
    zacc = jnp.zeros((ACC_ROWS // 16, HC), f32)

    ke_p, qe_p, ve_p = _s1(k_p, q_a, v_p, sp, dp)
    w_p2a, exw_p2a = _s2(ke_p, qe_p, ve_p, d8_p, s16, r8)
    ke_a, qe_a, ve_a = _s1(k_a, q_p, v_a, sa, da)
    w_a2p, exw_a2p = _s2(ke_a, qe_a, ve_a, d8_a, s16, r8)

    agg_a2, den_a2 = _s3(w_p2a, exw_p2a, il_p, d3_p, bd_p, zacc)
    agg_p2, den_p2 = _s3(w_a2p, exw_a2p, il_a, d3_a, bd_a, zacc)

    bp = jax.nn.sigmoid(skip_p).reshape(1, 1)
    ba = jax.nn.sigmoid(skip_a).reshape(1, 1)
    out_p = _post(agg_p2[0], agg_p2[1],
                  den_p2[0].reshape(N_PAD, DH), den_p2[1].reshape(N_PAD, DH),
                  xn_p, Wo_p.T, b2(bo_p), r8, bp, b2(prelu_w))
    out_a = _post(agg_a2[0], agg_a2[1],
                  den_a2[0].reshape(N_PAD, DH), den_a2[1].reshape(N_PAD, DH),
                  xn_a, Wo_a.T, b2(bo_a), r8, ba, b2(prelu_w))
    return out_p[:N_NODE], out_a[:N_NODE]
